# Initial kernel scaffold; baseline (speedup 1.0000x reference)
import jax
import jax.numpy as jnp
from jax.experimental import pallas as pl


def kernel(x_query, x_product, edge_click, edge_qr, edge_rclick, edge_rqr, W_self_l0_click, W_neigh_l0_click, b_l0_click, W_self_l0_qr, W_neigh_l0_qr, b_l0_qr, W_self_l0_rclick, W_neigh_l0_rclick, b_l0_rclick, W_self_l0_rqr, W_neigh_l0_rqr, b_l0_rqr, W_self_l1_click, W_neigh_l1_click, b_l1_click, W_self_l1_qr, W_neigh_l1_qr, b_l1_qr, W_self_l1_rclick, W_neigh_l1_rclick, b_l1_rclick, W_self_l1_rqr, W_neigh_l1_rqr, b_l1_rqr):
    raise NotImplementedError("write your pallas kernel here")


# speedup vs baseline: 2.4145x; 2.4145x over previous
import functools
from functools import partial

import jax
import jax.numpy as jnp
from jax import lax
from jax.experimental import pallas as pl
from jax.experimental.pallas import tpu as pltpu
from jax.experimental.pallas import tpu_sc as plsc

N = 50000
D = 128
E = 150000
NC, NS, L = 2, 16, 16
CH = 32
NCHUNK = D // CH
B = 128
NBATCH = 74
EPAD = NS * NBATCH * B
EPW = NBATCH * B
ACC_R = 50048
RPT = ACC_R // NS
DUMMY = N

_mesh = plsc.VectorSubcoreMesh(core_axis_name="c", subcore_axis_name="s")


def _fill2d(ref, rows, value):
    vec = jnp.full((L,), value, dtype=jnp.float32)

    def body(i, _):
        for j in range(CH // L):
            ref[i, pl.ds(j * L, L)] = vec
        return 0

    lax.fori_loop(0, rows, body, 0)


def _zero_acc(acc, zbuf, sid):
    row0 = sid * RPT
    nfull = RPT // B
    rem = RPT - nfull * B

    def body(i, _):
        pltpu.sync_copy(zbuf, acc.at[pl.ds(row0 + i * B, B)])
        return 0

    lax.fori_loop(0, nfull, body, 0)
    pltpu.sync_copy(zbuf.at[pl.ds(0, rem)], acc.at[pl.ds(row0 + nfull * B, rem)])


def _agg_body(h2, srcp, dstp, out, acc, zbuf, src_v, gidx_v, dst_v, rows_v, gsem):
    cid = lax.axis_index("c")
    sid = lax.axis_index("s")
    ebase = sid * EPW

    _fill2d(zbuf, B, 0.0)
    for rnd in range(2):
        chunk = cid * 2 + rnd
        _zero_acc(acc, zbuf, sid)
        plsc.subcore_barrier()

        def body(it, _):
            base = ebase + it * B
            pltpu.sync_copy(srcp.at[pl.ds(base, B)], src_v)
            pltpu.sync_copy(dstp.at[pl.ds(base, B)], dst_v)
            for j in range(B // L):
                v = src_v[pl.ds(j * L, L)]
                gidx_v[pl.ds(j * L, L)] = v * NCHUNK + chunk
            pltpu.async_copy(h2.at[gidx_v], rows_v, gsem).wait()
            pltpu.sync_copy(rows_v, acc.at[dst_v], add=True)
            return 0

        lax.fori_loop(0, NBATCH, body, 0)
        plsc.subcore_barrier()
        row0 = sid * RPT
        pltpu.sync_copy(acc.at[pl.ds(row0, RPT)],
                        out.at[chunk, pl.ds(row0, RPT)])
        plsc.subcore_barrier()


def _sc_aggregate(h, srcp, dstp):
    h2 = h.reshape(N * NCHUNK, CH)
    kern = pl.kernel(
        _agg_body,
        out_type=jax.ShapeDtypeStruct((NCHUNK, ACC_R, CH), jnp.float32),
        mesh=_mesh,
        scratch_types=[
            pltpu.VMEM_SHARED((ACC_R, CH), jnp.float32),
            pltpu.VMEM((B, CH), jnp.float32),
            pltpu.VMEM((B,), jnp.int32),
            pltpu.VMEM((B,), jnp.int32),
            pltpu.VMEM((B,), jnp.int32),
            pltpu.VMEM((B, CH), jnp.float32),
            pltpu.SemaphoreType.DMA,
        ],
        compiler_params=pltpu.CompilerParams(use_tc_tiling_on_sc=False),
    )
    return kern(h2, srcp, dstp)


def _deg_half(dstp, out, acc, obuf, dst_v, sid):
    _fill2d(obuf, B, 0.0)
    _zero_acc(acc, obuf, sid)
    plsc.subcore_barrier()
    _fill2d(obuf, B, 1.0)
    ebase = sid * EPW

    def body(it, _):
        pltpu.sync_copy(dstp.at[pl.ds(ebase + it * B, B)], dst_v)
        pltpu.sync_copy(obuf, acc.at[dst_v], add=True)
        return 0

    lax.fori_loop(0, NBATCH, body, 0)
    plsc.subcore_barrier()
    row0 = sid * RPT
    pltpu.sync_copy(acc.at[pl.ds(row0, RPT)], out.at[pl.ds(row0, RPT)])


def _deg_body(dstp_a, dstp_b, out_a, out_b, acc, obuf, dst_v):
    cid = lax.axis_index("c")
    sid = lax.axis_index("s")

    @pl.when(cid == 0)
    def _():
        _deg_half(dstp_a, out_a, acc, obuf, dst_v, sid)

    @pl.when(cid == 1)
    def _():
        _deg_half(dstp_b, out_b, acc, obuf, dst_v, sid)


def _sc_degrees(dstp_a, dstp_b):
    kern = pl.kernel(
        _deg_body,
        out_type=(jax.ShapeDtypeStruct((ACC_R, CH), jnp.float32),
                  jax.ShapeDtypeStruct((ACC_R, CH), jnp.float32)),
        mesh=_mesh,
        scratch_types=[
            pltpu.VMEM_SHARED((ACC_R, CH), jnp.float32),
            pltpu.VMEM((B, CH), jnp.float32),
            pltpu.VMEM((B,), jnp.int32),
        ],
        compiler_params=pltpu.CompilerParams(use_tc_tiling_on_sc=False),
    )
    return kern(dstp_a, dstp_b)


ROWS_BLK = 1000
GRID = N // ROWS_BLK


def _mm_body(h_ref, a1_ref, d1_ref, a2_ref, d2_ref, ws_ref, w1_ref, w2_ref,
             b_ref, o_ref, *, relu):
    r1 = 1.0 / jnp.maximum(d1_ref[:, 0:1], 1.0)
    r2 = 1.0 / jnp.maximum(d2_ref[:, 0:1], 1.0)
    acc = jnp.dot(h_ref[...], ws_ref[...], preferred_element_type=jnp.float32)
    for c in range(NCHUNK):
        acc += jnp.dot(a1_ref[c] * r1, w1_ref[c * CH:(c + 1) * CH, :],
                       preferred_element_type=jnp.float32)
        acc += jnp.dot(a2_ref[c] * r2, w2_ref[c * CH:(c + 1) * CH, :],
                       preferred_element_type=jnp.float32)
    acc += b_ref[...]
    if relu:
        acc = jnp.maximum(acc, 0.0)
    o_ref[...] = acc


def _tc_combine(h, agg1, deg1, agg2, deg2, ws, w1, w2, b, relu):
    row_spec = pl.BlockSpec((ROWS_BLK, D), lambda i: (i, 0))
    agg_spec = pl.BlockSpec((NCHUNK, ROWS_BLK, CH), lambda i: (0, i, 0))
    deg_spec = pl.BlockSpec((ROWS_BLK, CH), lambda i: (i, 0))
    full = pl.BlockSpec((D, D), lambda i: (0, 0))
    bspec = pl.BlockSpec((1, D), lambda i: (0, 0))
    return pl.pallas_call(
        partial(_mm_body, relu=relu),
        grid=(GRID,),
        in_specs=[row_spec, agg_spec, deg_spec, agg_spec, deg_spec,
                  full, full, full, bspec],
        out_specs=row_spec,
        out_shape=jax.ShapeDtypeStruct((N, D), jnp.float32),
    )(h, agg1, deg1, agg2, deg2, ws, w1, w2, b.reshape(1, D))


def _pad_edges(edge):
    npad = EPAD - E
    src = jnp.concatenate([edge[0], jnp.zeros((npad,), jnp.int32)])
    dst = jnp.concatenate([edge[1], jnp.full((npad,), DUMMY, jnp.int32)])
    return src, dst


@jax.jit
def kernel(x_query, x_product, edge_click, edge_qr, edge_rclick, edge_rqr,
           W_self_l0_click, W_neigh_l0_click, b_l0_click,
           W_self_l0_qr, W_neigh_l0_qr, b_l0_qr,
           W_self_l0_rclick, W_neigh_l0_rclick, b_l0_rclick,
           W_self_l0_rqr, W_neigh_l0_rqr, b_l0_rqr,
           W_self_l1_click, W_neigh_l1_click, b_l1_click,
           W_self_l1_qr, W_neigh_l1_qr, b_l1_qr,
           W_self_l1_rclick, W_neigh_l1_rclick, b_l1_rclick,
           W_self_l1_rqr, W_neigh_l1_rqr, b_l1_rqr):
    sc_click = _pad_edges(edge_click)
    sc_qr = _pad_edges(edge_qr)
    sc_rclick = _pad_edges(edge_rclick)
    sc_rqr = _pad_edges(edge_rqr)

    deg_click, deg_qr = _sc_degrees(sc_click[1], sc_qr[1])
    deg_rclick, deg_rqr = _sc_degrees(sc_rclick[1], sc_rqr[1])

    def layer(hq, hp, l, relu):
        def prm(kind, r):
            return {
                ("W_self", 0, "click"): W_self_l0_click,
                ("W_neigh", 0, "click"): W_neigh_l0_click,
                ("b", 0, "click"): b_l0_click,
                ("W_self", 0, "qr"): W_self_l0_qr,
                ("W_neigh", 0, "qr"): W_neigh_l0_qr,
                ("b", 0, "qr"): b_l0_qr,
                ("W_self", 0, "rclick"): W_self_l0_rclick,
                ("W_neigh", 0, "rclick"): W_neigh_l0_rclick,
                ("b", 0, "rclick"): b_l0_rclick,
                ("W_self", 0, "rqr"): W_self_l0_rqr,
                ("W_neigh", 0, "rqr"): W_neigh_l0_rqr,
                ("b", 0, "rqr"): b_l0_rqr,
                ("W_self", 1, "click"): W_self_l1_click,
                ("W_neigh", 1, "click"): W_neigh_l1_click,
                ("b", 1, "click"): b_l1_click,
                ("W_self", 1, "qr"): W_self_l1_qr,
                ("W_neigh", 1, "qr"): W_neigh_l1_qr,
                ("b", 1, "qr"): b_l1_qr,
                ("W_self", 1, "rclick"): W_self_l1_rclick,
                ("W_neigh", 1, "rclick"): W_neigh_l1_rclick,
                ("b", 1, "rclick"): b_l1_rclick,
                ("W_self", 1, "rqr"): W_self_l1_rqr,
                ("W_neigh", 1, "rqr"): W_neigh_l1_rqr,
                ("b", 1, "rqr"): b_l1_rqr,
            }[(kind, l, r)]

        agg_click = _sc_aggregate(hq, *sc_click)
        agg_qr = _sc_aggregate(hq, *sc_qr)
        agg_rclick = _sc_aggregate(hp, *sc_rclick)
        agg_rqr = _sc_aggregate(hp, *sc_rqr)

        hp_new = _tc_combine(
            hp, agg_click, deg_click, agg_qr, deg_qr,
            prm("W_self", "click") + prm("W_self", "qr"),
            prm("W_neigh", "click"), prm("W_neigh", "qr"),
            prm("b", "click") + prm("b", "qr"), relu)
        hq_new = _tc_combine(
            hq, agg_rclick, deg_rclick, agg_rqr, deg_rqr,
            prm("W_self", "rclick") + prm("W_self", "rqr"),
            prm("W_neigh", "rclick"), prm("W_neigh", "rqr"),
            prm("b", "rclick") + prm("b", "rqr"), relu)
        return hq_new, hp_new

    hq, hp = layer(x_query, x_product, 0, True)
    hq, hp = layer(hq, hp, 1, False)
    return hq, hp
